# baseline (speedup 1.0000x reference)
import functools

import jax
import jax.numpy as jnp
from jax import lax
from jax.experimental import pallas as pl
from jax.experimental.pallas import tpu as pltpu
from jax.experimental.pallas import tpu_sc as plsc

_N = 10000
_E = 320000
_F = 128
_NP = 10240
_PAD = 10000
_NC = 2
_NS = 16
_NW = _NC * _NS
_NCH = 80
_EPT = _NCH * 128
_EPAD = _NW * _EPT
_RPT = _NP // _NS
_NCHW = _EPAD // _NS // 128
_HF = _F // 2


def _ring_pipeline(tab, src_v, dst_v, rows_v, acc, semg, sems, nch):
    def fire_g(ch, k):
        pltpu.async_copy(tab.at[src_v.at[ch]], rows_v.at[k], semg[k])

    def drain_g(ch, k):
        pltpu.make_async_copy(tab.at[src_v.at[ch]], rows_v.at[k],
                              semg[k]).wait()

    def fire_s(ch, k):
        pltpu.async_copy(rows_v.at[k], acc.at[dst_v.at[ch]], sems[k],
                         add=True)

    def drain_s(ch, k):
        pltpu.make_async_copy(rows_v.at[k], acc.at[dst_v.at[ch]],
                              sems[k]).wait()

    for k in range(3):
        fire_g(k, k)

    def body(i, _):
        base = 4 * i
        for k in range(4):
            ch = base + k
            j = (k + 3) % 4
            drain_g(ch, k)
            fire_s(ch, k)

            @pl.when(ch > 0)
            def _():
                drain_s(ch - 1, j)

            @pl.when(ch + 3 < nch)
            def _():
                fire_g(ch + 3, j)

        return 0

    lax.fori_loop(0, nch // 4, body, 0)
    drain_s(nch - 1, 3)

def _sc_propagate_narrow(table, srcp, dstp, width):
    mesh = plsc.VectorSubcoreMesh(core_axis_name="c", subcore_axis_name="s")

    @functools.partial(
        pl.kernel,
        out_type=jax.ShapeDtypeStruct((_NC * _NP, width), jnp.float32),
        mesh=mesh,
        compiler_params=pltpu.CompilerParams(use_tc_tiling_on_sc=False),
        scratch_types=[
            pltpu.VMEM((_NCH, 128), jnp.int32),
            pltpu.VMEM((_NCH, 128), jnp.int32),
            pltpu.VMEM((4, 128, width), jnp.float32),
            pltpu.VMEM((128, width), jnp.float32),
            pltpu.VMEM_SHARED((_NP, width), jnp.float32),
        ] + [pltpu.SemaphoreType.DMA] * 8,
    )
    def k(tab_hbm, src_hbm, dst_hbm, out_hbm, src_v, dst_v, rows_v, zb_v, acc,
          *sems):
        c = lax.axis_index("c")
        s = lax.axis_index("s")
        wid = c * _NS + s
        row0 = s * _RPT

        pltpu.sync_copy(src_hbm.at[wid], src_v)
        pltpu.sync_copy(dst_hbm.at[wid], dst_v)

        zero16 = jnp.zeros((16,), jnp.float32)

        def _zrow(i, _):
            for j in range(width // 16):
                zb_v[i, pl.ds(j * 16, 16)] = zero16
            return 0

        lax.fori_loop(0, 128, _zrow, 0)
        for kk in range(_RPT // 128):
            pltpu.sync_copy(zb_v, acc.at[pl.ds(row0 + kk * 128, 128)])

        plsc.subcore_barrier()
        _ring_pipeline(tab_hbm, src_v, dst_v, rows_v, acc,
                       sems[:4], sems[4:], _NCH)

        plsc.subcore_barrier()
        pltpu.sync_copy(acc.at[pl.ds(row0, _RPT)],
                        out_hbm.at[pl.ds(c * _NP + row0, _RPT)])

    return k(table, srcp, dstp)


def _sc_propagate_wide(table2, srcp, dstp):
    mesh = plsc.VectorSubcoreMesh(core_axis_name="c", subcore_axis_name="s")

    @functools.partial(
        pl.kernel,
        out_type=jax.ShapeDtypeStruct((_NC * _NP, _HF), jnp.float32),
        mesh=mesh,
        compiler_params=pltpu.CompilerParams(use_tc_tiling_on_sc=False),
        scratch_types=[
            pltpu.VMEM((2, 8, 128), jnp.int32),
            pltpu.VMEM((2, 8, 128), jnp.int32),
            pltpu.VMEM((8, 128, _HF), jnp.float32),
            pltpu.VMEM_SHARED((_NP, _HF), jnp.float32),
        ] + [pltpu.SemaphoreType.DMA] * 17,
    )
    def k(tab_hbm, src_hbm, dst_hbm, out_hbm, sidx_v, didx_v, rows_v, acc,
          *sems):
        c = lax.axis_index("c")
        s = lax.axis_index("s")
        row0 = s * _RPT
        tab = tab_hbm.at[c]
        srct = src_hbm.at[s]
        dstt = dst_hbm.at[s]
        semg = sems[:8]
        semsc = sems[8:16]
        semi = sems[16]
        nsb = _NCHW // 8

        zero16 = jnp.zeros((16,), jnp.float32)

        def _zrow(i, _):
            for j in range(_HF // 16):
                rows_v[0, i, pl.ds(j * 16, 16)] = zero16
            return 0

        lax.fori_loop(0, 128, _zrow, 0)
        for kk in range(_RPT // 128):
            pltpu.sync_copy(rows_v.at[0], acc.at[pl.ds(row0 + kk * 128, 128)])

        pltpu.sync_copy(srct.at[pl.ds(0, 8)], sidx_v.at[0])
        pltpu.sync_copy(dstt.at[pl.ds(0, 8)], didx_v.at[0])
        pltpu.async_copy(srct.at[pl.ds(8, 8)], sidx_v.at[1], semi)
        pltpu.async_copy(dstt.at[pl.ds(8, 8)], didx_v.at[1], semi)
        for k0 in range(8):
            pltpu.async_copy(tab.at[sidx_v.at[0].at[k0]], rows_v.at[k0],
                             semg[k0])
        plsc.subcore_barrier()

        def _drain_s(ib, j):
            pltpu.make_async_copy(rows_v.at[j], acc.at[didx_v.at[ib].at[j]],
                                  semsc[j]).wait()

        def _fire_next_g(i, nb, j):
            @pl.when(i < nsb - 1)
            def _():
                pltpu.async_copy(tab.at[sidx_v.at[nb].at[j]], rows_v.at[j],
                                 semg[j])

        def body(i, _):
            ib = lax.rem(i, 2)
            nb = 1 - ib

            @pl.when(i < nsb - 1)
            def _():
                pltpu.make_async_copy(srct.at[pl.ds((i + 1) * 8, 8)],
                                      sidx_v.at[nb], semi).wait()
                pltpu.make_async_copy(dstt.at[pl.ds((i + 1) * 8, 8)],
                                      didx_v.at[nb], semi).wait()

            for k in range(8):
                pltpu.make_async_copy(tab.at[sidx_v.at[ib].at[k]],
                                      rows_v.at[k], semg[k]).wait()
                pltpu.async_copy(rows_v.at[k], acc.at[didx_v.at[ib].at[k]],
                                 semsc[k], add=True)
                if k >= 2:
                    _drain_s(ib, k - 2)
                    _fire_next_g(i, nb, k - 2)
            for j in (6, 7):
                _drain_s(ib, j)
                _fire_next_g(i, nb, j)

            @pl.when(i < nsb - 2)
            def _():
                pltpu.async_copy(srct.at[pl.ds((i + 2) * 8, 8)],
                                 sidx_v.at[ib], semi)
                pltpu.async_copy(dstt.at[pl.ds((i + 2) * 8, 8)],
                                 didx_v.at[ib], semi)

            return 0

        lax.fori_loop(0, nsb, body, 0)

        plsc.subcore_barrier()
        pltpu.sync_copy(acc.at[pl.ds(row0, _RPT)],
                        out_hbm.at[pl.ds(c * _NP + row0, _RPT)])

    return k(table2, srcp, dstp)


def _tc_edge_prep(src_pad, dst_pad):
    def body(s_ref, d_ref, o_ref):
        s = s_ref[...]
        o_ref[...] = jnp.where(s == d_ref[...], _PAD, s)

    rows = _EPAD // 128
    return pl.pallas_call(
        body,
        out_shape=jax.ShapeDtypeStruct((rows, 128), jnp.int32),
        grid=(4,),
        in_specs=[pl.BlockSpec((rows // 4, 128), lambda i: (i, 0)),
                  pl.BlockSpec((rows // 4, 128), lambda i: (i, 0))],
        out_specs=pl.BlockSpec((rows // 4, 128), lambda i: (i, 0)),
    )(src_pad, dst_pad)


def _tc_prep1(dega, degb, x16f):
    bm = 128
    rows = _NP * 16 // 128
    lim = _N * 16 // 128

    def body(a_ref, b_ref, x_ref, dis_ref, g_ref):
        i = pl.program_id(0)
        deg = a_ref[...] + b_ref[...]
        dis = jnp.where(deg > 0, lax.rsqrt(deg), 0.0)
        r = lax.broadcasted_iota(jnp.int32, (bm, 128), 0) + i * bm
        dis = jnp.where(r < lim, dis, 0.0)
        dis_ref[...] = dis
        g_ref[...] = dis * x_ref[...]

    return pl.pallas_call(
        body,
        out_shape=[jax.ShapeDtypeStruct((rows, 128), jnp.float32)] * 2,
        grid=(rows // bm,),
        in_specs=[pl.BlockSpec((bm, 128), lambda i: (i, 0))] * 3,
        out_specs=[pl.BlockSpec((bm, 128), lambda i: (i, 0))] * 2,
    )(dega, degb, x16f)


def _tc_combine_narrow(q0f, disf):
    bm = 128
    rows = _NP * 16 // 128
    nb = rows // bm

    def body(a_ref, b_ref, d_ref, t_ref, g_ref):
        d = d_ref[...]
        t1 = -d * (a_ref[...] + b_ref[...])
        t_ref[...] = t1
        g_ref[...] = d * t1

    return pl.pallas_call(
        body,
        out_shape=[jax.ShapeDtypeStruct((rows, 128), jnp.float32)] * 2,
        grid=(nb,),
        in_specs=[pl.BlockSpec((bm, 128), lambda i: (i, 0)),
                  pl.BlockSpec((bm, 128), lambda i: (i + nb, 0)),
                  pl.BlockSpec((bm, 128), lambda i: (i, 0))],
        out_specs=[pl.BlockSpec((bm, 128), lambda i: (i, 0))] * 2,
    )(q0f, q0f, disf)


def _tc_layer1(x16, t1_16, q1, dis16, w11, b11):
    bm = 512
    nb = _NP // bm

    def body(x_ref, t_ref, a_ref, b_ref, d_ref, w_ref, bias_ref, h_ref, g_ref):
        xc = x_ref[...][:, 0:1]
        t1 = t_ref[...][:, 0:1]
        d = d_ref[...][:, 0:1]
        tx2 = -2.0 * d * (a_ref[...][:, 0:1] + b_ref[...][:, 0:1]) - xc
        w = w_ref[...]
        out = xc * w[0:1, :] + t1 * w[1:2, :] + tx2 * w[2:3, :] + bias_ref[...]
        h = jnp.maximum(out, 0.0)
        h_ref[...] = h
        g = d * h
        g_ref[0] = g[:, :_HF]
        g_ref[1] = g[:, _HF:]

    return pl.pallas_call(
        body,
        out_shape=[jax.ShapeDtypeStruct((_NP, _F), jnp.float32),
                   jax.ShapeDtypeStruct((2, _NP, _HF), jnp.float32)],
        grid=(_NP // bm,),
        in_specs=[
            pl.BlockSpec((bm, 16), lambda i: (i, 0)),
            pl.BlockSpec((bm, 16), lambda i: (i, 0)),
            pl.BlockSpec((bm, 16), lambda i: (i, 0)),
            pl.BlockSpec((bm, 16), lambda i: (i + nb, 0)),
            pl.BlockSpec((bm, 16), lambda i: (i, 0)),
            pl.BlockSpec((3, 128), lambda i: (0, 0)),
            pl.BlockSpec((1, 128), lambda i: (0, 0)),
        ],
        out_specs=[pl.BlockSpec((bm, _F), lambda i: (i, 0)),
                   pl.BlockSpec((2, bm, _HF), lambda i: (0, i, 0))],
    )(x16, t1_16, q1, q1, dis16, w11, b11)


def _tc_combine_wide(qa, qb, dis16):
    bm = 512

    def body(a_ref, b_ref, d_ref, t_ref, g_ref):
        d = d_ref[...][:, 0:1]
        t1 = -d * jnp.concatenate([a_ref[...], b_ref[...]], axis=1)
        t_ref[...] = t1
        g = d * t1
        g_ref[0] = g[:, :_HF]
        g_ref[1] = g[:, _HF:]

    return pl.pallas_call(
        body,
        out_shape=[jax.ShapeDtypeStruct((_NP, _F), jnp.float32),
                   jax.ShapeDtypeStruct((2, _NP, _HF), jnp.float32)],
        grid=(_NP // bm,),
        in_specs=[
            pl.BlockSpec((bm, _HF), lambda i: (i, 0)),
            pl.BlockSpec((bm, _HF), lambda i: (i, 0)),
            pl.BlockSpec((bm, 16), lambda i: (i, 0)),
        ],
        out_specs=[pl.BlockSpec((bm, _F), lambda i: (i, 0)),
                   pl.BlockSpec((2, bm, _HF), lambda i: (0, i, 0))],
    )(qa, qb, dis16)


def _tc_cheb_out(h_in, t1, r, dis16, W, b, res, do_relu, emit_table):
    bm = 512

    def body(*refs):
        if res is not None:
            (h_ref, t_ref, a_ref, b_ref, d_ref, w_ref, bias_ref, r_ref,
             *outs) = refs
        else:
            h_ref, t_ref, a_ref, b_ref, d_ref, w_ref, bias_ref, *outs = refs
            r_ref = None
        d = d_ref[...][:, 0:1]
        h = h_ref[...]
        rr = jnp.concatenate([a_ref[...], b_ref[...]], axis=1)
        tx2 = -2.0 * d * rr - h
        w = w_ref[...]
        out = jnp.dot(h, w[0], preferred_element_type=jnp.float32, precision=lax.Precision.HIGHEST)
        out += jnp.dot(t_ref[...], w[1], preferred_element_type=jnp.float32, precision=lax.Precision.HIGHEST)
        out += jnp.dot(tx2, w[2], preferred_element_type=jnp.float32, precision=lax.Precision.HIGHEST)
        out += bias_ref[...]
        if r_ref is not None:
            out += r_ref[...]
        if do_relu:
            out = jnp.maximum(out, 0.0)
        outs[0][...] = out
        if emit_table:
            g = d * out
            outs[1][0] = g[:, :_HF]
            outs[1][1] = g[:, _HF:]

    in_specs = [
        pl.BlockSpec((bm, _F), lambda i: (i, 0)),
        pl.BlockSpec((bm, _F), lambda i: (i, 0)),
        pl.BlockSpec((bm, _HF), lambda i: (i, 0)),
        pl.BlockSpec((bm, _HF), lambda i: (i, 0)),
        pl.BlockSpec((bm, 16), lambda i: (i, 0)),
        pl.BlockSpec((3, _F, _F), lambda i: (0, 0, 0)),
        pl.BlockSpec((1, _F), lambda i: (0, 0)),
    ]
    args = [h_in, t1, r[:_NP], r[_NP:], dis16, W, b]
    if res is not None:
        in_specs.append(pl.BlockSpec((bm, _F), lambda i: (i, 0)))
        args.append(res)
    out_shape = [jax.ShapeDtypeStruct((_NP, _F), jnp.float32)]
    out_specs = [pl.BlockSpec((bm, _F), lambda i: (i, 0))]
    if emit_table:
        out_shape.append(jax.ShapeDtypeStruct((2, _NP, _HF), jnp.float32))
        out_specs.append(pl.BlockSpec((2, bm, _HF), lambda i: (0, i, 0)))
    return pl.pallas_call(
        body,
        out_shape=out_shape,
        grid=(_NP // bm,),
        in_specs=in_specs,
        out_specs=out_specs,
    )(*args)


def _tc_mlp(h, w1, b1, w2, b2, w3, b3):
    bm = 512

    def body(h_ref, w1_ref, b1_ref, w2_ref, b2_ref, w3_ref, b3_ref, o_ref):
        z = jnp.dot(h_ref[...], w1_ref[...], preferred_element_type=jnp.float32, precision=lax.Precision.HIGHEST)
        z = jnp.maximum(z + b1_ref[...], 0.0)
        z = jnp.dot(z, w2_ref[...], preferred_element_type=jnp.float32, precision=lax.Precision.HIGHEST)
        z = jnp.maximum(z + b2_ref[...], 0.0)
        z = jnp.dot(z, w3_ref[...], preferred_element_type=jnp.float32, precision=lax.Precision.HIGHEST)
        o_ref[...] = jnp.maximum(z + b3_ref[...], 0.0)

    H = 1024
    return pl.pallas_call(
        body,
        out_shape=jax.ShapeDtypeStruct((_NP, _F), jnp.float32),
        grid=(_NP // bm,),
        in_specs=[
            pl.BlockSpec((bm, _F), lambda i: (i, 0)),
            pl.BlockSpec((_F, H), lambda i: (0, 0)),
            pl.BlockSpec((1, H), lambda i: (0, 0)),
            pl.BlockSpec((H, H), lambda i: (0, 0)),
            pl.BlockSpec((1, H), lambda i: (0, 0)),
            pl.BlockSpec((H, _F), lambda i: (0, 0)),
            pl.BlockSpec((1, _F), lambda i: (0, 0)),
        ],
        out_specs=pl.BlockSpec((bm, _F), lambda i: (i, 0)),
    )(h, w1, b1, w2, b2, w3, b3)


def kernel(x, edge_index, batch, W11, b11, W12, b12, W13, b13, W21, b21,
           W22, b22, W31, b31, L1w, L1b, L2w, L2b, L3w, L3b):
    src = edge_index[0]
    dst = edge_index[1]
    src_pad = jnp.full((_EPAD,), _PAD, jnp.int32).at[:_E].set(src)
    dst_pad = jnp.full((_EPAD,), _PAD, jnp.int32).at[:_E].set(dst)
    srcp = _tc_edge_prep(src_pad.reshape(_EPAD // 128, 128),
                         dst_pad.reshape(_EPAD // 128, 128))
    srcp3 = srcp.reshape(_NW, _NCH, 128)
    dstp3 = dst_pad.reshape(_NW, _NCH, 128)
    srcpW = srcp.reshape(_NS, _NCHW, 128)
    dstpW = dst_pad.reshape(_NS, _NCHW, 128)

    ones_tab = jnp.concatenate([
        jnp.ones((_N, 16), jnp.float32),
        jnp.zeros((_NP - _N, 16), jnp.float32)])
    degp = _sc_propagate_narrow(ones_tab, srcp3, srcp3, 16)
    dega = degp[:_NP].reshape(_NP * 16 // 128, 128)
    degb = degp[_NP:].reshape(_NP * 16 // 128, 128)

    xp = jnp.zeros((_NP, 1), jnp.float32).at[:_N].set(x)
    x16 = jnp.broadcast_to(xp, (_NP, 16))
    x16f = x16.reshape(_NP * 16 // 128, 128)

    disf, g0f = _tc_prep1(dega, degb, x16f)
    dis16 = disf.reshape(_NP, 16)

    q0 = _sc_propagate_narrow(g0f.reshape(_NP, 16), srcp3, dstp3, 16)
    t1f, g1f = _tc_combine_narrow(q0.reshape(-1, 128), disf)
    q1 = _sc_propagate_narrow(g1f.reshape(_NP, 16), srcp3, dstp3, 16)
    h1, g = _tc_layer1(x16, t1f.reshape(_NP, 16), q1,
                       dis16, W11.reshape(3, 128), b11.reshape(1, _F))

    def cheb_wide(h_in, g_in, W, b, res, do_relu, emit_table):
        q = _sc_propagate_wide(g_in, srcpW, dstpW)
        t1w, g2 = _tc_combine_wide(q[:_NP], q[_NP:], dis16)
        r = _sc_propagate_wide(g2, srcpW, dstpW)
        return _tc_cheb_out(h_in, t1w, r, dis16, W,
                            b.reshape(1, _F), res, do_relu, emit_table)

    h2, g = cheb_wide(h1, g, W12, b12, None, True, True)
    hA, g = cheb_wide(h2, g, W13, b13, h1, False, True)
    h4, g = cheb_wide(hA, g, W21, b21, None, True, True)
    hB, g = cheb_wide(h4, g, W22, b22, hA, False, True)
    (h6,) = cheb_wide(hB, g, W31, b31, None, True, False)

    out = _tc_mlp(h6, L1w, L1b.reshape(1, 1024), L2w, L2b.reshape(1, 1024),
                  jnp.pad(L3w, ((0, 0), (0, _F - 10))),
                  jnp.pad(L3b, (0, _F - 10)).reshape(1, _F))
    return out[:_N, :10]


# speedup vs baseline: 1.0554x; 1.0554x over previous
import functools

import jax
import jax.numpy as jnp
from jax import lax
from jax.experimental import pallas as pl
from jax.experimental.pallas import tpu as pltpu
from jax.experimental.pallas import tpu_sc as plsc

_N = 10000
_E = 320000
_F = 128
_NP = 10240
_PAD = 10000
_NC = 2
_NS = 16
_NW = _NC * _NS
_NCH = 80
_EPT = _NCH * 128
_EPAD = _NW * _EPT
_RPT = _NP // _NS
_NCHW = _EPAD // _NS // 128
_HF = _F // 2


def _ring_pipeline(tab, src_v, dst_v, rows_v, acc, semg, sems, nch):
    def fire_g(ch, k):
        pltpu.async_copy(tab.at[src_v.at[ch]], rows_v.at[k], semg[k])

    def drain_g(ch, k):
        pltpu.make_async_copy(tab.at[src_v.at[ch]], rows_v.at[k],
                              semg[k]).wait()

    def fire_s(ch, k):
        pltpu.async_copy(rows_v.at[k], acc.at[dst_v.at[ch]], sems[k],
                         add=True)

    def drain_s(ch, k):
        pltpu.make_async_copy(rows_v.at[k], acc.at[dst_v.at[ch]],
                              sems[k]).wait()

    for k in range(3):
        fire_g(k, k)

    def body(i, _):
        base = 4 * i
        for k in range(4):
            ch = base + k
            j = (k + 3) % 4
            drain_g(ch, k)
            fire_s(ch, k)

            @pl.when(ch > 0)
            def _():
                drain_s(ch - 1, j)

            @pl.when(ch + 3 < nch)
            def _():
                fire_g(ch + 3, j)

        return 0

    lax.fori_loop(0, nch // 4, body, 0)
    drain_s(nch - 1, 3)

def _sc_propagate_narrow(table, srcp, dstp, width):
    mesh = plsc.VectorSubcoreMesh(core_axis_name="c", subcore_axis_name="s")

    @functools.partial(
        pl.kernel,
        out_type=jax.ShapeDtypeStruct((_NC * _NP, width), jnp.float32),
        mesh=mesh,
        compiler_params=pltpu.CompilerParams(use_tc_tiling_on_sc=False),
        scratch_types=[
            pltpu.VMEM((_NCH, 128), jnp.int32),
            pltpu.VMEM((_NCH, 128), jnp.int32),
            pltpu.VMEM((4, 128, width), jnp.float32),
            pltpu.VMEM((128, width), jnp.float32),
            pltpu.VMEM_SHARED((_NP, width), jnp.float32),
        ] + [pltpu.SemaphoreType.DMA] * 8,
    )
    def k(tab_hbm, src_hbm, dst_hbm, out_hbm, src_v, dst_v, rows_v, zb_v, acc,
          *sems):
        c = lax.axis_index("c")
        s = lax.axis_index("s")
        wid = c * _NS + s
        row0 = s * _RPT

        pltpu.sync_copy(src_hbm.at[wid], src_v)
        pltpu.sync_copy(dst_hbm.at[wid], dst_v)

        zero16 = jnp.zeros((16,), jnp.float32)

        def _zrow(i, _):
            for j in range(width // 16):
                zb_v[i, pl.ds(j * 16, 16)] = zero16
            return 0

        lax.fori_loop(0, 128, _zrow, 0)
        for kk in range(_RPT // 128):
            pltpu.sync_copy(zb_v, acc.at[pl.ds(row0 + kk * 128, 128)])

        plsc.subcore_barrier()
        _ring_pipeline(tab_hbm, src_v, dst_v, rows_v, acc,
                       sems[:4], sems[4:], _NCH)

        plsc.subcore_barrier()
        pltpu.sync_copy(acc.at[pl.ds(row0, _RPT)],
                        out_hbm.at[pl.ds(c * _NP + row0, _RPT)])

    return k(table, srcp, dstp)


def _sc_propagate_wide(table2, srcp, dstp):
    mesh = plsc.VectorSubcoreMesh(core_axis_name="c", subcore_axis_name="s")

    @functools.partial(
        pl.kernel,
        out_type=jax.ShapeDtypeStruct((_NC * _NP, _HF), jnp.float32),
        mesh=mesh,
        compiler_params=pltpu.CompilerParams(use_tc_tiling_on_sc=False),
        scratch_types=[
            pltpu.VMEM((2, 8, 128), jnp.int32),
            pltpu.VMEM((2, 8, 128), jnp.int32),
            pltpu.VMEM((8, 128, _HF), jnp.float32),
            pltpu.VMEM_SHARED((_NP, _HF), jnp.float32),
        ] + [pltpu.SemaphoreType.DMA] * 17,
    )
    def k(tab_hbm, src_hbm, dst_hbm, out_hbm, sidx_v, didx_v, rows_v, acc,
          *sems):
        c = lax.axis_index("c")
        s = lax.axis_index("s")
        row0 = s * _RPT
        tab = tab_hbm.at[c]
        srct = src_hbm.at[s]
        dstt = dst_hbm.at[s]
        semg = sems[:8]
        semsc = sems[8:16]
        semi = sems[16]
        nsb = _NCHW // 8

        zero16 = jnp.zeros((16,), jnp.float32)

        def _zrow(i, _):
            for j in range(_HF // 16):
                rows_v[0, i, pl.ds(j * 16, 16)] = zero16
            return 0

        lax.fori_loop(0, 128, _zrow, 0)
        for kk in range(_RPT // 128):
            pltpu.sync_copy(rows_v.at[0], acc.at[pl.ds(row0 + kk * 128, 128)])

        pltpu.sync_copy(srct.at[pl.ds(0, 8)], sidx_v.at[0])
        pltpu.sync_copy(dstt.at[pl.ds(0, 8)], didx_v.at[0])
        pltpu.async_copy(srct.at[pl.ds(8, 8)], sidx_v.at[1], semi)
        pltpu.async_copy(dstt.at[pl.ds(8, 8)], didx_v.at[1], semi)
        for k0 in range(8):
            pltpu.async_copy(tab.at[sidx_v.at[0].at[k0]], rows_v.at[k0],
                             semg[k0])
        plsc.subcore_barrier()

        def _drain_s(ib, j):
            pltpu.make_async_copy(rows_v.at[j], acc.at[didx_v.at[ib].at[j]],
                                  semsc[j]).wait()

        def _fire_next_g(i, nb, j):
            @pl.when(i < nsb - 1)
            def _():
                pltpu.async_copy(tab.at[sidx_v.at[nb].at[j]], rows_v.at[j],
                                 semg[j])

        def body(i, _):
            ib = lax.rem(i, 2)
            nb = 1 - ib

            @pl.when(i < nsb - 1)
            def _():
                pltpu.make_async_copy(srct.at[pl.ds((i + 1) * 8, 8)],
                                      sidx_v.at[nb], semi).wait()
                pltpu.make_async_copy(dstt.at[pl.ds((i + 1) * 8, 8)],
                                      didx_v.at[nb], semi).wait()

            for k in range(8):
                pltpu.make_async_copy(tab.at[sidx_v.at[ib].at[k]],
                                      rows_v.at[k], semg[k]).wait()
                pltpu.async_copy(rows_v.at[k], acc.at[didx_v.at[ib].at[k]],
                                 semsc[k], add=True)
                if k >= 2:
                    _drain_s(ib, k - 2)
                    _fire_next_g(i, nb, k - 2)
            for j in (6, 7):
                _drain_s(ib, j)
                _fire_next_g(i, nb, j)

            @pl.when(i < nsb - 2)
            def _():
                pltpu.async_copy(srct.at[pl.ds((i + 2) * 8, 8)],
                                 sidx_v.at[ib], semi)
                pltpu.async_copy(dstt.at[pl.ds((i + 2) * 8, 8)],
                                 didx_v.at[ib], semi)

            return 0

        lax.fori_loop(0, nsb, body, 0)

        plsc.subcore_barrier()
        pltpu.sync_copy(acc.at[pl.ds(row0, _RPT)],
                        out_hbm.at[pl.ds(c * _NP + row0, _RPT)])

    return k(table2, srcp, dstp)


def _tc_edge_prep(src_pad, dst_pad):
    def body(s_ref, d_ref, o_ref):
        s = s_ref[...]
        o_ref[...] = jnp.where(s == d_ref[...], _PAD, s)

    rows = _EPAD // 128
    return pl.pallas_call(
        body,
        out_shape=jax.ShapeDtypeStruct((rows, 128), jnp.int32),
        grid=(4,),
        in_specs=[pl.BlockSpec((rows // 4, 128), lambda i: (i, 0)),
                  pl.BlockSpec((rows // 4, 128), lambda i: (i, 0))],
        out_specs=pl.BlockSpec((rows // 4, 128), lambda i: (i, 0)),
    )(src_pad, dst_pad)


def _tc_prep1(dega, degb, x16f):
    bm = 128
    rows = _NP * 16 // 128
    lim = _N * 16 // 128

    def body(a_ref, b_ref, x_ref, dis_ref, g_ref):
        i = pl.program_id(0)
        deg = a_ref[...] + b_ref[...]
        dis = jnp.where(deg > 0, lax.rsqrt(deg), 0.0)
        r = lax.broadcasted_iota(jnp.int32, (bm, 128), 0) + i * bm
        dis = jnp.where(r < lim, dis, 0.0)
        dis_ref[...] = dis
        g_ref[...] = dis * x_ref[...]

    return pl.pallas_call(
        body,
        out_shape=[jax.ShapeDtypeStruct((rows, 128), jnp.float32)] * 2,
        grid=(rows // bm,),
        in_specs=[pl.BlockSpec((bm, 128), lambda i: (i, 0))] * 3,
        out_specs=[pl.BlockSpec((bm, 128), lambda i: (i, 0))] * 2,
    )(dega, degb, x16f)


def _tc_combine_narrow(q0f, disf):
    bm = 128
    rows = _NP * 16 // 128
    nb = rows // bm

    def body(a_ref, b_ref, d_ref, t_ref, g_ref):
        d = d_ref[...]
        t1 = -d * (a_ref[...] + b_ref[...])
        t_ref[...] = t1
        g_ref[...] = d * t1

    return pl.pallas_call(
        body,
        out_shape=[jax.ShapeDtypeStruct((rows, 128), jnp.float32)] * 2,
        grid=(nb,),
        in_specs=[pl.BlockSpec((bm, 128), lambda i: (i, 0)),
                  pl.BlockSpec((bm, 128), lambda i: (i + nb, 0)),
                  pl.BlockSpec((bm, 128), lambda i: (i, 0))],
        out_specs=[pl.BlockSpec((bm, 128), lambda i: (i, 0))] * 2,
    )(q0f, q0f, disf)


def _tc_layer1(x16, t1_16, q1, dis16, w11, b11):
    bm = 512
    nb = _NP // bm

    def body(x_ref, t_ref, a_ref, b_ref, d_ref, w_ref, bias_ref, h_ref, g_ref):
        xc = x_ref[...][:, 0:1]
        t1 = t_ref[...][:, 0:1]
        d = d_ref[...][:, 0:1]
        tx2 = -2.0 * d * (a_ref[...][:, 0:1] + b_ref[...][:, 0:1]) - xc
        w = w_ref[...]
        out = xc * w[0:1, :] + t1 * w[1:2, :] + tx2 * w[2:3, :] + bias_ref[...]
        h = jnp.maximum(out, 0.0)
        h_ref[...] = h
        g = d * h
        g_ref[0] = g[:, :_HF]
        g_ref[1] = g[:, _HF:]

    return pl.pallas_call(
        body,
        out_shape=[jax.ShapeDtypeStruct((_NP, _F), jnp.float32),
                   jax.ShapeDtypeStruct((2, _NP, _HF), jnp.float32)],
        grid=(_NP // bm,),
        in_specs=[
            pl.BlockSpec((bm, 16), lambda i: (i, 0)),
            pl.BlockSpec((bm, 16), lambda i: (i, 0)),
            pl.BlockSpec((bm, 16), lambda i: (i, 0)),
            pl.BlockSpec((bm, 16), lambda i: (i + nb, 0)),
            pl.BlockSpec((bm, 16), lambda i: (i, 0)),
            pl.BlockSpec((3, 128), lambda i: (0, 0)),
            pl.BlockSpec((1, 128), lambda i: (0, 0)),
        ],
        out_specs=[pl.BlockSpec((bm, _F), lambda i: (i, 0)),
                   pl.BlockSpec((2, bm, _HF), lambda i: (0, i, 0))],
    )(x16, t1_16, q1, q1, dis16, w11, b11)


def _tc_combine_wide(qa, qb, dis16):
    bm = 512

    def body(a_ref, b_ref, d_ref, t_ref, g_ref):
        d = d_ref[...][:, 0:1]
        t1 = -d * jnp.concatenate([a_ref[...], b_ref[...]], axis=1)
        t_ref[...] = t1
        g = d * t1
        g_ref[0] = g[:, :_HF]
        g_ref[1] = g[:, _HF:]

    return pl.pallas_call(
        body,
        out_shape=[jax.ShapeDtypeStruct((_NP, _F), jnp.float32),
                   jax.ShapeDtypeStruct((2, _NP, _HF), jnp.float32)],
        grid=(_NP // bm,),
        in_specs=[
            pl.BlockSpec((bm, _HF), lambda i: (i, 0)),
            pl.BlockSpec((bm, _HF), lambda i: (i, 0)),
            pl.BlockSpec((bm, 16), lambda i: (i, 0)),
        ],
        out_specs=[pl.BlockSpec((bm, _F), lambda i: (i, 0)),
                   pl.BlockSpec((2, bm, _HF), lambda i: (0, i, 0))],
    )(qa, qb, dis16)


def _tc_cheb_out(h_in, t1, r, dis16, W, b, res, do_relu, emit_table):
    bm = 512

    def body(*refs):
        if res is not None:
            (h_ref, t_ref, a_ref, b_ref, d_ref, w_ref, bias_ref, r_ref,
             *outs) = refs
        else:
            h_ref, t_ref, a_ref, b_ref, d_ref, w_ref, bias_ref, *outs = refs
            r_ref = None
        d = d_ref[...][:, 0:1]
        h = h_ref[...]
        rr = jnp.concatenate([a_ref[...], b_ref[...]], axis=1)
        tx2 = -2.0 * d * rr - h
        w = w_ref[...]
        out = jnp.dot(h, w[0], preferred_element_type=jnp.float32)
        out += jnp.dot(t_ref[...], w[1], preferred_element_type=jnp.float32)
        out += jnp.dot(tx2, w[2], preferred_element_type=jnp.float32)
        out += bias_ref[...]
        if r_ref is not None:
            out += r_ref[...]
        if do_relu:
            out = jnp.maximum(out, 0.0)
        outs[0][...] = out
        if emit_table:
            g = d * out
            outs[1][0] = g[:, :_HF]
            outs[1][1] = g[:, _HF:]

    in_specs = [
        pl.BlockSpec((bm, _F), lambda i: (i, 0)),
        pl.BlockSpec((bm, _F), lambda i: (i, 0)),
        pl.BlockSpec((bm, _HF), lambda i: (i, 0)),
        pl.BlockSpec((bm, _HF), lambda i: (i, 0)),
        pl.BlockSpec((bm, 16), lambda i: (i, 0)),
        pl.BlockSpec((3, _F, _F), lambda i: (0, 0, 0)),
        pl.BlockSpec((1, _F), lambda i: (0, 0)),
    ]
    args = [h_in, t1, r[:_NP], r[_NP:], dis16, W, b]
    if res is not None:
        in_specs.append(pl.BlockSpec((bm, _F), lambda i: (i, 0)))
        args.append(res)
    out_shape = [jax.ShapeDtypeStruct((_NP, _F), jnp.float32)]
    out_specs = [pl.BlockSpec((bm, _F), lambda i: (i, 0))]
    if emit_table:
        out_shape.append(jax.ShapeDtypeStruct((2, _NP, _HF), jnp.float32))
        out_specs.append(pl.BlockSpec((2, bm, _HF), lambda i: (0, i, 0)))
    return pl.pallas_call(
        body,
        out_shape=out_shape,
        grid=(_NP // bm,),
        in_specs=in_specs,
        out_specs=out_specs,
    )(*args)


def _tc_mlp(h, w1, b1, w2, b2, w3, b3):
    bm = 512

    def body(h_ref, w1_ref, b1_ref, w2_ref, b2_ref, w3_ref, b3_ref, o_ref):
        z = jnp.dot(h_ref[...], w1_ref[...], preferred_element_type=jnp.float32)
        z = jnp.maximum(z + b1_ref[...], 0.0)
        z = jnp.dot(z, w2_ref[...], preferred_element_type=jnp.float32)
        z = jnp.maximum(z + b2_ref[...], 0.0)
        z = jnp.dot(z, w3_ref[...], preferred_element_type=jnp.float32)
        o_ref[...] = jnp.maximum(z + b3_ref[...], 0.0)

    H = 1024
    return pl.pallas_call(
        body,
        out_shape=jax.ShapeDtypeStruct((_NP, _F), jnp.float32),
        grid=(_NP // bm,),
        in_specs=[
            pl.BlockSpec((bm, _F), lambda i: (i, 0)),
            pl.BlockSpec((_F, H), lambda i: (0, 0)),
            pl.BlockSpec((1, H), lambda i: (0, 0)),
            pl.BlockSpec((H, H), lambda i: (0, 0)),
            pl.BlockSpec((1, H), lambda i: (0, 0)),
            pl.BlockSpec((H, _F), lambda i: (0, 0)),
            pl.BlockSpec((1, _F), lambda i: (0, 0)),
        ],
        out_specs=pl.BlockSpec((bm, _F), lambda i: (i, 0)),
    )(h, w1, b1, w2, b2, w3, b3)


def kernel(x, edge_index, batch, W11, b11, W12, b12, W13, b13, W21, b21,
           W22, b22, W31, b31, L1w, L1b, L2w, L2b, L3w, L3b):
    src = edge_index[0]
    dst = edge_index[1]
    src_pad = jnp.full((_EPAD,), _PAD, jnp.int32).at[:_E].set(src)
    dst_pad = jnp.full((_EPAD,), _PAD, jnp.int32).at[:_E].set(dst)
    srcp = _tc_edge_prep(src_pad.reshape(_EPAD // 128, 128),
                         dst_pad.reshape(_EPAD // 128, 128))
    srcp3 = srcp.reshape(_NW, _NCH, 128)
    dstp3 = dst_pad.reshape(_NW, _NCH, 128)
    srcpW = srcp.reshape(_NS, _NCHW, 128)
    dstpW = dst_pad.reshape(_NS, _NCHW, 128)

    ones_tab = jnp.concatenate([
        jnp.ones((_N, 16), jnp.float32),
        jnp.zeros((_NP - _N, 16), jnp.float32)])
    degp = _sc_propagate_narrow(ones_tab, srcp3, srcp3, 16)
    dega = degp[:_NP].reshape(_NP * 16 // 128, 128)
    degb = degp[_NP:].reshape(_NP * 16 // 128, 128)

    xp = jnp.zeros((_NP, 1), jnp.float32).at[:_N].set(x)
    x16 = jnp.broadcast_to(xp, (_NP, 16))
    x16f = x16.reshape(_NP * 16 // 128, 128)

    disf, g0f = _tc_prep1(dega, degb, x16f)
    dis16 = disf.reshape(_NP, 16)

    q0 = _sc_propagate_narrow(g0f.reshape(_NP, 16), srcp3, dstp3, 16)
    t1f, g1f = _tc_combine_narrow(q0.reshape(-1, 128), disf)
    q1 = _sc_propagate_narrow(g1f.reshape(_NP, 16), srcp3, dstp3, 16)
    h1, g = _tc_layer1(x16, t1f.reshape(_NP, 16), q1,
                       dis16, W11.reshape(3, 128), b11.reshape(1, _F))

    def cheb_wide(h_in, g_in, W, b, res, do_relu, emit_table):
        q = _sc_propagate_wide(g_in, srcpW, dstpW)
        t1w, g2 = _tc_combine_wide(q[:_NP], q[_NP:], dis16)
        r = _sc_propagate_wide(g2, srcpW, dstpW)
        return _tc_cheb_out(h_in, t1w, r, dis16, W,
                            b.reshape(1, _F), res, do_relu, emit_table)

    h2, g = cheb_wide(h1, g, W12, b12, None, True, True)
    hA, g = cheb_wide(h2, g, W13, b13, h1, False, True)
    h4, g = cheb_wide(hA, g, W21, b21, None, True, True)
    hB, g = cheb_wide(h4, g, W22, b22, hA, False, True)
    (h6,) = cheb_wide(hB, g, W31, b31, None, True, False)

    out = _tc_mlp(h6, L1w, L1b.reshape(1, 1024), L2w, L2b.reshape(1, 1024),
                  jnp.pad(L3w, ((0, 0), (0, _F - 10))),
                  jnp.pad(L3b, (0, _F - 10)).reshape(1, _F))
    return out[:_N, :10]
